# Initial kernel scaffold; baseline (speedup 1.0000x reference)
import jax
import jax.numpy as jnp
from jax.experimental import pallas as pl


def kernel(x_document, x_statute, x_section, x_claim, edge_index_cites, edge_index_contains, edge_index_references, params):
    raise NotImplementedError("write your pallas kernel here")


# speedup vs baseline: 18.6026x; 18.6026x over previous
import functools

import jax
import jax.numpy as jnp
from jax import lax
from jax.experimental import pallas as pl
from jax.experimental.pallas import tpu as pltpu
from jax.experimental.pallas import tpu_sc as plsc

F32 = jnp.float32
I32 = jnp.int32

NTYPES = ["document", "statute", "section", "claim"]
ERELS = [
    ("document", "cites", "statute", "cites", False),
    ("statute", "contains", "section", "contains", False),
    ("claim", "references", "document", "references", False),
    ("statute", "rev_cites", "document", "cites", True),
    ("section", "rev_contains", "statute", "contains", True),
    ("document", "rev_references", "claim", "references", True),
]
DST_RELS = {"document": [2, 3], "statute": [0, 4], "section": [1], "claim": [5]}

N = 10000
E = 160000
H = 128
NH = 4
HD = 32

R_BLK = 1000
GRID = N // R_BLK

NBLK = E // 128
NSUB = 16
BPT = (NBLK + NSUB - 1) // NSUB
STRIPE = 624
ZROWS = 104


def _rb(i):
    return (i, 0)


def _const(i):
    return (0, 0)


def _embed(x_raw, emb):
    ins = [x_raw[nt] for nt in NTYPES]
    ws = [emb[nt]["W"] for nt in NTYPES]
    bs = [emb[nt]["b"].reshape(1, H) for nt in NTYPES]

    def body(*refs):
        xs = refs[0:4]
        wr = refs[4:8]
        br = refs[8:12]
        outs = refs[12:16]
        for i in range(4):
            outs[i][...] = (
                jnp.dot(xs[i][...], wr[i][...], preferred_element_type=F32)
                + br[i][...]
            )

    return pl.pallas_call(
        body,
        grid=(GRID,),
        in_specs=(
            [pl.BlockSpec((R_BLK, 768), _rb)] * 4
            + [pl.BlockSpec((768, H), _const)] * 4
            + [pl.BlockSpec((1, H), _const)] * 4
        ),
        out_specs=[pl.BlockSpec((R_BLK, H), _rb)] * 4,
        out_shape=[jax.ShapeDtypeStruct((N, H), F32)] * 4,
    )(*ins, *ws, *bs)


def _wself(e_cites, e_contains, e_refs):
    def body(ec, eo, er, out):
        m_doc = jnp.maximum(jnp.max(ec[0:1, :]), jnp.max(er[1:2, :]))
        m_sta = jnp.maximum(jnp.max(ec[1:2, :]), jnp.max(eo[0:1, :]))
        m_sec = jnp.max(eo[1:2, :])
        m_clm = jnp.max(er[0:1, :])
        for i, m in enumerate([m_doc, m_sta, m_sec, m_clm]):
            out[i] = jnp.where(m > 0, 1.0, 0.0).astype(F32)

    return pl.pallas_call(
        body,
        in_specs=[pl.BlockSpec((2, E), _const)] * 3,
        grid=(1,),
        out_specs=pl.BlockSpec(memory_space=pltpu.SMEM),
        out_shape=jax.ShapeDtypeStruct((4,), F32),
    )(e_cites, e_contains, e_refs)


def _pre(x, conv):
    ws_e, vs16, vd16 = [], [], []
    for (s, r, d, ek, sw) in ERELS:
        p = conv[f"{s}__{r}__{d}"]
        vs = jnp.einsum("khj,hj->kh", p["Ws"].reshape(H, NH, HD), p["as"])
        vd = jnp.einsum("khj,hj->kh", p["Wd"].reshape(H, NH, HD), p["ad"])
        ws_e.append(p["Ws"])
        vs16.append(jnp.pad(vs, ((0, 0), (0, 12))))
        vd16.append(jnp.pad(vd, ((0, 0), (0, 12))))
    ws_s, bs_s = [], []
    for nt in NTYPES:
        p = conv[f"{nt}__self_{nt}__{nt}"]
        ws_s.append(p["Ws"])
        bs_s.append(p["b"].reshape(1, H))

    src_of = [ERELS[i][0] for i in range(6)]
    dst_of = [ERELS[i][2] for i in range(6)]

    def body(*refs):
        xb = dict(zip(NTYPES, refs[0:4]))
        wse = refs[4:10]
        vsr = refs[10:16]
        vdr = refs[16:22]
        wss = refs[22:26]
        bss = refs[26:30]
        o_xs = refs[30:36]
        o_as = refs[36:42]
        o_ad = refs[42:48]
        o_self = refs[48:52]
        for i in range(6):
            xsrc = xb[src_of[i]][...]
            xdst = xb[dst_of[i]][...]
            o_xs[i][...] = jnp.dot(xsrc, wse[i][...], preferred_element_type=F32)
            o_as[i][...] = jnp.dot(xsrc, vsr[i][...], preferred_element_type=F32)
            o_ad[i][...] = jnp.dot(xdst, vdr[i][...], preferred_element_type=F32)
        for i, nt in enumerate(NTYPES):
            o_self[i][...] = (
                jnp.dot(xb[nt][...], wss[i][...], preferred_element_type=F32)
                + bss[i][...]
            )

    outs = pl.pallas_call(
        body,
        grid=(GRID,),
        in_specs=(
            [pl.BlockSpec((R_BLK, H), _rb)] * 4
            + [pl.BlockSpec((H, H), _const)] * 6
            + [pl.BlockSpec((H, 16), _const)] * 12
            + [pl.BlockSpec((H, H), _const)] * 4
            + [pl.BlockSpec((1, H), _const)] * 4
        ),
        out_specs=(
            [pl.BlockSpec((R_BLK, H), _rb)] * 6
            + [pl.BlockSpec((R_BLK, 16), _rb)] * 12
            + [pl.BlockSpec((R_BLK, H), _rb)] * 4
        ),
        out_shape=(
            [jax.ShapeDtypeStruct((N, H), F32)] * 6
            + [jax.ShapeDtypeStruct((N, 16), F32)] * 12
            + [jax.ShapeDtypeStruct((N, H), F32)] * 4
        ),
    )(*[x[nt] for nt in NTYPES], *ws_e, *vs16, *vd16, *ws_s, *bs_s)
    xs = outs[0:6]
    asc = outs[6:12]
    adc = outs[12:18]
    selfo = outs[18:22]
    return xs, asc, adc, selfo


def _sc_edges(xs, asc, adc, srcs, dsts):
    mesh = plsc.VectorSubcoreMesh(core_axis_name="c", subcore_axis_name="s")

    def body(*refs):
        xsr = refs[0:6]
        ascr = refs[6:12]
        adcr = refs[12:18]
        srcr = refs[18:24]
        dstr = refs[24:30]
        numr = refs[30:36]
        denr = refs[36:42]
        (idx_s, idx_d, idx_dg, rows, asr, adr, exb, zbuf, zden,
         acc, dacc, sem, sem2, sem3) = refs[42:]

        core = lax.axis_index("c")
        sub = lax.axis_index("s")

        def zb_row(j, _):
            def zb_col(k, _):
                zbuf[j, pl.ds(k * 16, 16)] = jnp.zeros((16,), F32)
                return 0

            lax.fori_loop(0, H // 16, zb_col, 0)
            zden[j, pl.ds(0, 16)] = jnp.zeros((16,), F32)
            return 0

        lax.fori_loop(0, ZROWS, zb_row, 0)

        for r in range(6):
            @pl.when(core == r % 2)
            def _do_rel(r=r):
                for t in range(STRIPE // ZROWS):
                    base = sub * STRIPE + t * ZROWS
                    pltpu.sync_copy(zbuf, acc.at[pl.ds(base, ZROWS)])
                    pltpu.sync_copy(zden, dacc.at[pl.ds(base, ZROWS)])

                @pl.when(sub == NSUB - 1)
                def _ztail():
                    pltpu.sync_copy(zbuf.at[pl.ds(0, 16)],
                                    acc.at[pl.ds(NSUB * STRIPE, 16)])
                    pltpu.sync_copy(zden.at[pl.ds(0, 16)],
                                    dacc.at[pl.ds(NSUB * STRIPE, 16)])

                plsc.subcore_barrier()

                nblk_t = jnp.minimum(BPT, NBLK - sub * BPT)

                def blk(k, _):
                    bg = sub * BPT + k
                    ebase = bg * 128
                    pltpu.sync_copy(srcr[r].at[pl.ds(ebase, 128)], idx_s)
                    pltpu.sync_copy(dstr[r].at[pl.ds(ebase, 128)], idx_dg)
                    pltpu.sync_copy(dstr[r].at[pl.ds(ebase, 128)], idx_d.at[0])
                    c1 = pltpu.async_copy(ascr[r].at[idx_s], asr, sem)
                    c2 = pltpu.async_copy(adcr[r].at[idx_dg], adr, sem2)
                    c3 = pltpu.async_copy(xsr[r].at[idx_s], rows, sem3)
                    c1.wait()
                    c2.wait()
                    c3.wait()

                    def edge(e, _):
                        v = asr[e] + adr[e]
                        v = jnp.maximum(v, 0.2 * v)
                        ev = jnp.exp(v)
                        exb[e] = ev
                        for h in range(NH):
                            bh = jnp.full((16,), ev[h], F32)
                            for q in range(2):
                                sl = pl.ds(h * 32 + q * 16, 16)
                                rows[e, sl] = rows[e, sl] * bh
                        return 0

                    lax.fori_loop(0, 128, edge, 0)
                    pltpu.sync_copy(rows, acc.at[idx_d.at[0]], add=True)
                    pltpu.sync_copy(exb, dacc.at[idx_d.at[0]], add=True)
                    return 0

                lax.fori_loop(0, nblk_t, blk, 0)
                plsc.subcore_barrier()

                rbase = sub * STRIPE
                pltpu.sync_copy(acc.at[pl.ds(rbase, STRIPE)],
                                numr[r].at[pl.ds(rbase, STRIPE)])
                pltpu.sync_copy(dacc.at[pl.ds(rbase, STRIPE)],
                                denr[r].at[pl.ds(rbase, STRIPE)])

                @pl.when(sub == NSUB - 1)
                def _ctail():
                    pltpu.sync_copy(acc.at[pl.ds(NSUB * STRIPE, 16)],
                                    numr[r].at[pl.ds(NSUB * STRIPE, 16)])
                    pltpu.sync_copy(dacc.at[pl.ds(NSUB * STRIPE, 16)],
                                    denr[r].at[pl.ds(NSUB * STRIPE, 16)])

    f = pl.kernel(
        body,
        mesh=mesh,
        compiler_params=pltpu.CompilerParams(use_tc_tiling_on_sc=False),
        out_type=(
            [jax.ShapeDtypeStruct((N, H), F32)] * 6
            + [jax.ShapeDtypeStruct((N, 16), F32)] * 6
        ),
        scratch_types=[
            pltpu.VMEM((128,), I32),
            pltpu.VMEM((1, 128), I32),
            pltpu.VMEM((128,), I32),
            pltpu.VMEM((128, H), F32),
            pltpu.VMEM((128, 16), F32),
            pltpu.VMEM((128, 16), F32),
            pltpu.VMEM((128, 16), F32),
            pltpu.VMEM((ZROWS, H), F32),
            pltpu.VMEM((ZROWS, 16), F32),
            pltpu.VMEM_SHARED((N, H), F32),
            pltpu.VMEM_SHARED((N, 16), F32),
            pltpu.SemaphoreType.DMA,
            pltpu.SemaphoreType.DMA,
            pltpu.SemaphoreType.DMA,
        ],
    )
    outs = f(*xs, *asc, *adc, *srcs, *dsts)
    return outs[0:6], outs[6:12]


def _post(nums, dens, selfo, wself, conv, lnp):
    brels = []
    for (s, r, d, ek, sw) in ERELS:
        brels.append(conv[f"{s}__{r}__{d}"]["b"].reshape(1, H))
    lng = [lnp[nt]["g"].reshape(1, H) for nt in NTYPES]
    lnb = [lnp[nt]["b"].reshape(1, H) for nt in NTYPES]

    def body(*refs):
        numr = refs[0:6]
        denr = refs[6:12]
        selfr = refs[12:16]
        br = refs[16:22]
        gr = refs[22:26]
        b2r = refs[26:30]
        wr = refs[30]
        outs = refs[31:35]
        row = lax.broadcasted_iota(I32, (NH, H), 0)
        col = lax.broadcasted_iota(I32, (NH, H), 1) // HD
        expand = (row == col).astype(F32)
        for i, nt in enumerate(NTYPES):
            w = wr[i]
            tot = wr[i] * selfr[i][...]
            for rel in DST_RELS[nt]:
                dex = (
                    jnp.dot(denr[rel][:, 0:4], expand[:, :],
                            preferred_element_type=F32)
                    + 1e-16
                )
                tot = tot + numr[rel][...] / dex + br[rel][...]
            cnt = jnp.float32(len(DST_RELS[nt])) + w
            hh = jnp.maximum(tot / cnt, 0.0)
            mu = jnp.mean(hh, axis=-1, keepdims=True)
            var = jnp.mean((hh - mu) ** 2, axis=-1, keepdims=True)
            outs[i][...] = (
                (hh - mu) / jnp.sqrt(var + 1e-5) * gr[i][...] + b2r[i][...]
            )

    outs = pl.pallas_call(
        body,
        grid=(GRID,),
        in_specs=(
            [pl.BlockSpec((R_BLK, H), _rb)] * 6
            + [pl.BlockSpec((R_BLK, 16), _rb)] * 6
            + [pl.BlockSpec((R_BLK, H), _rb)] * 4
            + [pl.BlockSpec((1, H), _const)] * 14
            + [pl.BlockSpec(memory_space=pltpu.SMEM)]
        ),
        out_specs=[pl.BlockSpec((R_BLK, H), _rb)] * 4,
        out_shape=[jax.ShapeDtypeStruct((N, H), F32)] * 4,
    )(*nums, *dens, *selfo, *brels, *lng, *lnb, wself)
    return dict(zip(NTYPES, outs))


def _heads(claim, headp):
    names = ["citation_validity", "relevance_score", "coherence_score"]
    w1 = [headp[n]["W1"] for n in names]
    b1 = [headp[n]["b1"].reshape(1, H) for n in names]
    w2 = [jnp.pad(headp[n]["W2"], ((0, 0), (0, 7))) for n in names]
    b2 = [jnp.pad(headp[n]["b2"], (0, 7)).reshape(1, 8) for n in names]

    def body(c, *refs):
        w1r = refs[0:3]
        b1r = refs[3:6]
        w2r = refs[6:9]
        b2r = refs[9:12]
        outs = refs[12:15]
        cb = c[...]
        for i in range(3):
            hh = jnp.maximum(
                jnp.dot(cb, w1r[i][...], preferred_element_type=F32)
                + b1r[i][...],
                0.0,
            )
            z = jnp.dot(hh, w2r[i][...], preferred_element_type=F32) + b2r[i][...]
            outs[i][...] = 1.0 / (1.0 + jnp.exp(-z))

    outs = pl.pallas_call(
        body,
        grid=(GRID,),
        in_specs=(
            [pl.BlockSpec((R_BLK, H), _rb)]
            + [pl.BlockSpec((H, H), _const)] * 3
            + [pl.BlockSpec((1, H), _const)] * 3
            + [pl.BlockSpec((H, 8), _const)] * 3
            + [pl.BlockSpec((1, 8), _const)] * 3
        ),
        out_specs=[pl.BlockSpec((R_BLK, 8), _rb)] * 3,
        out_shape=[jax.ShapeDtypeStruct((N, 8), F32)] * 3,
    )(claim, *w1, *b1, *w2, *b2)
    return [o[:, 0:1] for o in outs]


def _sc_edges_xla(xs, asc, adc, srcs, dsts):
    nums, dens = [], []
    for r in range(6):
        al = asc[r][srcs[r]] + adc[r][dsts[r]]
        al = jnp.maximum(al, 0.2 * al)
        ex = jnp.exp(al)
        dens.append(jax.ops.segment_sum(ex, dsts[r], num_segments=N))
        msg = xs[r][srcs[r]].reshape(-1, NH, HD) * ex[:, 0:4][:, :, None]
        nums.append(jax.ops.segment_sum(msg, dsts[r],
                                        num_segments=N).reshape(N, H))
    return nums, dens


def kernel(x_document, x_statute, x_section, x_claim,
           edge_index_cites, edge_index_contains, edge_index_references,
           params):
    earr = {"cites": edge_index_cites, "contains": edge_index_contains,
            "references": edge_index_references}
    srcs, dsts = [], []
    for (s, r, d, ek, sw) in ERELS:
        ei = earr[ek]
        srcs.append(ei[1] if sw else ei[0])
        dsts.append(ei[0] if sw else ei[1])

    x_raw = {"document": x_document, "statute": x_statute,
             "section": x_section, "claim": x_claim}
    xs0 = _embed(x_raw, params["emb"])
    x = dict(zip(NTYPES, xs0))
    wself = _wself(edge_index_cites, edge_index_contains,
                   edge_index_references)

    for layer in range(2):
        conv = params["conv"][layer]
        xs, asc, adc, selfo = _pre(x, conv)
        nums, dens = _sc_edges(xs, asc, adc, srcs, dsts)
        x = _post(nums, dens, selfo, wself, conv, params["ln"][layer])

    claim = x["claim"]
    h1, h2, h3 = _heads(claim, params["heads"])
    return (claim, h1, h2, h3)


# speedup vs baseline: 33.4668x; 1.7990x over previous
import functools

import jax
import jax.numpy as jnp
from jax import lax
from jax.experimental import pallas as pl
from jax.experimental.pallas import tpu as pltpu
from jax.experimental.pallas import tpu_sc as plsc

F32 = jnp.float32
I32 = jnp.int32

NTYPES = ["document", "statute", "section", "claim"]
ERELS = [
    ("document", "cites", "statute", "cites", False),
    ("statute", "contains", "section", "contains", False),
    ("claim", "references", "document", "references", False),
    ("statute", "rev_cites", "document", "cites", True),
    ("section", "rev_contains", "statute", "contains", True),
    ("document", "rev_references", "claim", "references", True),
]
DST_RELS = {"document": [2, 3], "statute": [0, 4], "section": [1], "claim": [5]}

N = 10000
E = 160000
H = 128
NH = 4
HD = 32

R_BLK = 1000
GRID = N // R_BLK

NBLK = E // 128
NSUB = 16
BPT = 78
STRIPE = 624


def _rb(i):
    return (i, 0)


def _const(i):
    return (0, 0)


def _embed(x_raw, emb):
    ins = [x_raw[nt] for nt in NTYPES]
    ws = [emb[nt]["W"] for nt in NTYPES]
    bs = [emb[nt]["b"].reshape(1, H) for nt in NTYPES]

    def body(*refs):
        xs = refs[0:4]
        wr = refs[4:8]
        br = refs[8:12]
        outs = refs[12:16]
        for i in range(4):
            outs[i][...] = (
                jnp.dot(xs[i][...], wr[i][...], preferred_element_type=F32)
                + br[i][...]
            )

    return pl.pallas_call(
        body,
        grid=(GRID,),
        in_specs=(
            [pl.BlockSpec((R_BLK, 768), _rb)] * 4
            + [pl.BlockSpec((768, H), _const)] * 4
            + [pl.BlockSpec((1, H), _const)] * 4
        ),
        out_specs=[pl.BlockSpec((R_BLK, H), _rb)] * 4,
        out_shape=[jax.ShapeDtypeStruct((N, H), F32)] * 4,
    )(*ins, *ws, *bs)


def _wself(e_cites, e_contains, e_refs):
    def body(ec, eo, er, out):
        m_doc = jnp.maximum(jnp.max(ec[0:1, :]), jnp.max(er[1:2, :]))
        m_sta = jnp.maximum(jnp.max(ec[1:2, :]), jnp.max(eo[0:1, :]))
        m_sec = jnp.max(eo[1:2, :])
        m_clm = jnp.max(er[0:1, :])
        for i, m in enumerate([m_doc, m_sta, m_sec, m_clm]):
            out[i] = jnp.where(m > 0, 1.0, 0.0).astype(F32)

    return pl.pallas_call(
        body,
        in_specs=[pl.BlockSpec((2, E), _const)] * 3,
        grid=(1,),
        out_specs=pl.BlockSpec(memory_space=pltpu.SMEM),
        out_shape=jax.ShapeDtypeStruct((4,), F32),
    )(e_cites, e_contains, e_refs)


def _pre(x, conv):
    ws_e, vs16, vd16 = [], [], []
    for (s, r, d, ek, sw) in ERELS:
        p = conv[f"{s}__{r}__{d}"]
        vs = jnp.einsum("khj,hj->kh", p["Ws"].reshape(H, NH, HD), p["as"])
        vd = jnp.einsum("khj,hj->kh", p["Wd"].reshape(H, NH, HD), p["ad"])
        ws_e.append(p["Ws"])
        vs16.append(jnp.pad(vs, ((0, 0), (0, 12))))
        vd16.append(jnp.pad(vd, ((0, 0), (0, 12))))
    ws_s, bs_s = [], []
    for nt in NTYPES:
        p = conv[f"{nt}__self_{nt}__{nt}"]
        ws_s.append(p["Ws"])
        bs_s.append(p["b"].reshape(1, H))

    src_of = [ERELS[i][0] for i in range(6)]
    dst_of = [ERELS[i][2] for i in range(6)]

    def body(*refs):
        xb = dict(zip(NTYPES, refs[0:4]))
        wse = refs[4:10]
        vsr = refs[10:16]
        vdr = refs[16:22]
        wss = refs[22:26]
        bss = refs[26:30]
        o_xs = refs[30:36]
        o_as = refs[36:42]
        o_ad = refs[42:48]
        o_self = refs[48:52]
        for i in range(6):
            xsrc = xb[src_of[i]][...]
            xdst = xb[dst_of[i]][...]
            o_xs[i][...] = jnp.dot(xsrc, wse[i][...], preferred_element_type=F32)
            o_as[i][...] = jnp.dot(xsrc, vsr[i][...], preferred_element_type=F32)
            o_ad[i][...] = jnp.dot(xdst, vdr[i][...], preferred_element_type=F32)
        for i, nt in enumerate(NTYPES):
            o_self[i][...] = (
                jnp.dot(xb[nt][...], wss[i][...], preferred_element_type=F32)
                + bss[i][...]
            )

    outs = pl.pallas_call(
        body,
        grid=(GRID,),
        in_specs=(
            [pl.BlockSpec((R_BLK, H), _rb)] * 4
            + [pl.BlockSpec((H, H), _const)] * 6
            + [pl.BlockSpec((H, 16), _const)] * 12
            + [pl.BlockSpec((H, H), _const)] * 4
            + [pl.BlockSpec((1, H), _const)] * 4
        ),
        out_specs=(
            [pl.BlockSpec((R_BLK, H), _rb)] * 6
            + [pl.BlockSpec((R_BLK, 16), _rb)] * 12
            + [pl.BlockSpec((R_BLK, H), _rb)] * 4
        ),
        out_shape=(
            [jax.ShapeDtypeStruct((N, H), F32)] * 6
            + [jax.ShapeDtypeStruct((N, 16), F32)] * 12
            + [jax.ShapeDtypeStruct((N, H), F32)] * 4
        ),
    )(*[x[nt] for nt in NTYPES], *ws_e, *vs16, *vd16, *ws_s, *bs_s)
    xs = outs[0:6]
    asc = outs[6:12]
    adc = outs[12:18]
    selfo = outs[18:22]
    return xs, asc, adc, selfo


def _sc_edges(xs, asc, adc, srcs, dsts):
    mesh = plsc.VectorSubcoreMesh(core_axis_name="c", subcore_axis_name="s")
    z128 = jnp.zeros((STRIPE, H), F32)
    z16 = jnp.zeros((STRIPE, 16), F32)

    def body(*refs):
        xsr = refs[0:6]
        ascr = refs[6:12]
        adcr = refs[12:18]
        srcr = refs[18:24]
        dstr = refs[24:30]
        z128r, z16r = refs[30], refs[31]
        numr = refs[32:38]
        denr = refs[38:44]
        (idx_s, idx_d, rows, sbuf, adr,
         acc, dacc, sem_s, sem_a, sem_g, sem_sc) = refs[44:]

        core = lax.axis_index("c")
        sub = lax.axis_index("s")

        def ex_block(p):
            pb = p * 128

            def exe(i, _):
                for u in range(4):
                    e = i * 4 + u
                    v = sbuf[pb + e] + adr[e]
                    v = jnp.maximum(v, 0.2 * v)
                    sbuf[pb + e] = jnp.exp(v)
                return 0

            lax.fori_loop(0, 32, exe, 0)

        def mul_block(p):
            pb = p * 128

            def me(i, _):
                for u in range(4):
                    e = pb + i * 4 + u
                    ev = sbuf[e]
                    for h in range(NH):
                        bh = jnp.full((16,), ev[h], F32)
                        for q in range(2):
                            sl = pl.ds(h * 32 + q * 16, 16)
                            rows[e, sl] = rows[e, sl] * bh
                return 0

            lax.fori_loop(0, 32, me, 0)

        def stage_and_fire(r, b, p):
            pltpu.sync_copy(srcr[r].at[pl.ds(b * 128, 128)], idx_s)
            pltpu.sync_copy(dstr[r].at[pl.ds(b * 128, 128)], idx_d.at[p])
            pltpu.async_copy(ascr[r].at[idx_s],
                             sbuf.at[pl.ds(p * 128, 128)], sem_s)
            pltpu.async_copy(adcr[r].at[idx_d.at[p]], adr, sem_a)
            pltpu.async_copy(xsr[r].at[idx_s],
                             rows.at[pl.ds(p * 128, 128)], sem_g)

        def wait_g(p):
            pltpu.make_async_copy(z16r.at[pl.ds(0, 128)],
                                  sbuf.at[pl.ds(p * 128, 128)], sem_s).wait()
            pltpu.make_async_copy(z16r.at[pl.ds(0, 128)], adr, sem_a).wait()
            pltpu.make_async_copy(z128r.at[pl.ds(0, 128)],
                                  rows.at[pl.ds(p * 128, 128)], sem_g).wait()

        def fire_scatter(p):
            pltpu.async_copy(rows.at[pl.ds(p * 128, 128)],
                             acc.at[idx_d.at[p]], sem_sc, add=True)
            pltpu.async_copy(sbuf.at[pl.ds(p * 128, 128)],
                             dacc.at[idx_d.at[p]], sem_sc, add=True)

        def wait_scatter(p):
            pltpu.make_async_copy(rows.at[pl.ds(p * 128, 128)],
                                  acc.at[pl.ds(0, 128)], sem_sc).wait()
            pltpu.make_async_copy(sbuf.at[pl.ds(p * 128, 128)],
                                  dacc.at[pl.ds(0, 128)], sem_sc).wait()

        for r in range(6):
            @pl.when(core == r % 2)
            def _do_rel(r=r):
                rbase = sub * STRIPE
                pltpu.sync_copy(z128r, acc.at[pl.ds(rbase, STRIPE)])
                pltpu.sync_copy(z16r, dacc.at[pl.ds(rbase, STRIPE)])

                @pl.when(sub == NSUB - 1)
                def _ztail():
                    pltpu.sync_copy(z128r.at[pl.ds(0, 16)],
                                    acc.at[pl.ds(NSUB * STRIPE, 16)])
                    pltpu.sync_copy(z16r.at[pl.ds(0, 16)],
                                    dacc.at[pl.ds(NSUB * STRIPE, 16)])

                plsc.subcore_barrier()

                b0 = sub * BPT
                stage_and_fire(r, b0, 0)

                def pair(j, _):
                    bA = b0 + 2 * j
                    wait_g(0)
                    ex_block(0)

                    @pl.when(j > 0)
                    def _wsA():
                        wait_scatter(1)

                    stage_and_fire(r, bA + 1, 1)
                    mul_block(0)
                    fire_scatter(0)
                    wait_g(1)
                    ex_block(1)
                    wait_scatter(0)
                    stage_and_fire(r, bA + 2, 0)
                    mul_block(1)
                    fire_scatter(1)
                    return 0

                lax.fori_loop(0, BPT // 2, pair, 0)
                wait_scatter(1)
                wait_g(0)

                @pl.when(sub < 2)
                def _rem():
                    b = NSUB * BPT + sub
                    stage_and_fire(r, b, 0)
                    wait_g(0)
                    ex_block(0)
                    mul_block(0)
                    fire_scatter(0)
                    wait_scatter(0)

                plsc.subcore_barrier()

                pltpu.sync_copy(acc.at[pl.ds(rbase, STRIPE)],
                                numr[r].at[pl.ds(rbase, STRIPE)])
                pltpu.sync_copy(dacc.at[pl.ds(rbase, STRIPE)],
                                denr[r].at[pl.ds(rbase, STRIPE)])

                @pl.when(sub == NSUB - 1)
                def _ctail():
                    pltpu.sync_copy(acc.at[pl.ds(NSUB * STRIPE, 16)],
                                    numr[r].at[pl.ds(NSUB * STRIPE, 16)])
                    pltpu.sync_copy(dacc.at[pl.ds(NSUB * STRIPE, 16)],
                                    denr[r].at[pl.ds(NSUB * STRIPE, 16)])

    f = pl.kernel(
        body,
        mesh=mesh,
        compiler_params=pltpu.CompilerParams(use_tc_tiling_on_sc=False),
        out_type=(
            [jax.ShapeDtypeStruct((N, H), F32)] * 6
            + [jax.ShapeDtypeStruct((N, 16), F32)] * 6
        ),
        scratch_types=[
            pltpu.VMEM((128,), I32),
            pltpu.VMEM((2, 128), I32),
            pltpu.VMEM((2 * 128, H), F32),
            pltpu.VMEM((2 * 128, 16), F32),
            pltpu.VMEM((128, 16), F32),
            pltpu.VMEM_SHARED((N, H), F32),
            pltpu.VMEM_SHARED((N, 16), F32),
            pltpu.SemaphoreType.DMA,
            pltpu.SemaphoreType.DMA,
            pltpu.SemaphoreType.DMA,
            pltpu.SemaphoreType.DMA,
        ],
    )
    outs = f(*xs, *asc, *adc, *srcs, *dsts, z128, z16)
    return outs[0:6], outs[6:12]


def _post(nums, dens, selfo, wself, conv, lnp):
    brels = []
    for (s, r, d, ek, sw) in ERELS:
        brels.append(conv[f"{s}__{r}__{d}"]["b"].reshape(1, H))
    lng = [lnp[nt]["g"].reshape(1, H) for nt in NTYPES]
    lnb = [lnp[nt]["b"].reshape(1, H) for nt in NTYPES]

    def body(*refs):
        numr = refs[0:6]
        denr = refs[6:12]
        selfr = refs[12:16]
        br = refs[16:22]
        gr = refs[22:26]
        b2r = refs[26:30]
        wr = refs[30]
        outs = refs[31:35]
        row = lax.broadcasted_iota(I32, (NH, H), 0)
        col = lax.broadcasted_iota(I32, (NH, H), 1) // HD
        expand = (row == col).astype(F32)
        for i, nt in enumerate(NTYPES):
            w = wr[i]
            tot = wr[i] * selfr[i][...]
            for rel in DST_RELS[nt]:
                dex = (
                    jnp.dot(denr[rel][:, 0:4], expand[:, :],
                            preferred_element_type=F32)
                    + 1e-16
                )
                tot = tot + numr[rel][...] / dex + br[rel][...]
            cnt = jnp.float32(len(DST_RELS[nt])) + w
            hh = jnp.maximum(tot / cnt, 0.0)
            mu = jnp.mean(hh, axis=-1, keepdims=True)
            var = jnp.mean((hh - mu) ** 2, axis=-1, keepdims=True)
            outs[i][...] = (
                (hh - mu) / jnp.sqrt(var + 1e-5) * gr[i][...] + b2r[i][...]
            )

    outs = pl.pallas_call(
        body,
        grid=(GRID,),
        in_specs=(
            [pl.BlockSpec((R_BLK, H), _rb)] * 6
            + [pl.BlockSpec((R_BLK, 16), _rb)] * 6
            + [pl.BlockSpec((R_BLK, H), _rb)] * 4
            + [pl.BlockSpec((1, H), _const)] * 14
            + [pl.BlockSpec(memory_space=pltpu.SMEM)]
        ),
        out_specs=[pl.BlockSpec((R_BLK, H), _rb)] * 4,
        out_shape=[jax.ShapeDtypeStruct((N, H), F32)] * 4,
    )(*nums, *dens, *selfo, *brels, *lng, *lnb, wself)
    return dict(zip(NTYPES, outs))


def _heads(claim, headp):
    names = ["citation_validity", "relevance_score", "coherence_score"]
    w1 = [headp[n]["W1"] for n in names]
    b1 = [headp[n]["b1"].reshape(1, H) for n in names]
    w2 = [jnp.pad(headp[n]["W2"], ((0, 0), (0, 7))) for n in names]
    b2 = [jnp.pad(headp[n]["b2"], (0, 7)).reshape(1, 8) for n in names]

    def body(c, *refs):
        w1r = refs[0:3]
        b1r = refs[3:6]
        w2r = refs[6:9]
        b2r = refs[9:12]
        outs = refs[12:15]
        cb = c[...]
        for i in range(3):
            hh = jnp.maximum(
                jnp.dot(cb, w1r[i][...], preferred_element_type=F32)
                + b1r[i][...],
                0.0,
            )
            z = jnp.dot(hh, w2r[i][...], preferred_element_type=F32) + b2r[i][...]
            outs[i][...] = 1.0 / (1.0 + jnp.exp(-z))

    outs = pl.pallas_call(
        body,
        grid=(GRID,),
        in_specs=(
            [pl.BlockSpec((R_BLK, H), _rb)]
            + [pl.BlockSpec((H, H), _const)] * 3
            + [pl.BlockSpec((1, H), _const)] * 3
            + [pl.BlockSpec((H, 8), _const)] * 3
            + [pl.BlockSpec((1, 8), _const)] * 3
        ),
        out_specs=[pl.BlockSpec((R_BLK, 8), _rb)] * 3,
        out_shape=[jax.ShapeDtypeStruct((N, 8), F32)] * 3,
    )(claim, *w1, *b1, *w2, *b2)
    return [o[:, 0:1] for o in outs]


def _sc_edges_xla(xs, asc, adc, srcs, dsts):
    nums, dens = [], []
    for r in range(6):
        al = asc[r][srcs[r]] + adc[r][dsts[r]]
        al = jnp.maximum(al, 0.2 * al)
        ex = jnp.exp(al)
        dens.append(jax.ops.segment_sum(ex, dsts[r], num_segments=N))
        msg = xs[r][srcs[r]].reshape(-1, NH, HD) * ex[:, 0:4][:, :, None]
        nums.append(jax.ops.segment_sum(msg, dsts[r],
                                        num_segments=N).reshape(N, H))
    return nums, dens


def kernel(x_document, x_statute, x_section, x_claim,
           edge_index_cites, edge_index_contains, edge_index_references,
           params):
    earr = {"cites": edge_index_cites, "contains": edge_index_contains,
            "references": edge_index_references}
    srcs, dsts = [], []
    for (s, r, d, ek, sw) in ERELS:
        ei = earr[ek]
        srcs.append(ei[1] if sw else ei[0])
        dsts.append(ei[0] if sw else ei[1])

    x_raw = {"document": x_document, "statute": x_statute,
             "section": x_section, "claim": x_claim}
    xs0 = _embed(x_raw, params["emb"])
    x = dict(zip(NTYPES, xs0))
    wself = _wself(edge_index_cites, edge_index_contains,
                   edge_index_references)

    for layer in range(2):
        conv = params["conv"][layer]
        xs, asc, adc, selfo = _pre(x, conv)
        nums, dens = _sc_edges(xs, asc, adc, srcs, dsts)
        x = _post(nums, dens, selfo, wself, conv, params["ln"][layer])

    claim = x["claim"]
    h1, h2, h3 = _heads(claim, params["heads"])
    return (claim, h1, h2, h3)


# speedup vs baseline: 37.5933x; 1.1233x over previous
import functools

import jax
import jax.numpy as jnp
from jax import lax
from jax.experimental import pallas as pl
from jax.experimental.pallas import tpu as pltpu
from jax.experimental.pallas import tpu_sc as plsc

F32 = jnp.float32
I32 = jnp.int32

NTYPES = ["document", "statute", "section", "claim"]
ERELS = [
    ("document", "cites", "statute", "cites", False),
    ("statute", "contains", "section", "contains", False),
    ("claim", "references", "document", "references", False),
    ("statute", "rev_cites", "document", "cites", True),
    ("section", "rev_contains", "statute", "contains", True),
    ("document", "rev_references", "claim", "references", True),
]
DST_RELS = {"document": [2, 3], "statute": [0, 4], "section": [1], "claim": [5]}

N = 10000
E = 160000
H = 128
NH = 4
HD = 32

R_BLK = 1000
GRID = N // R_BLK

NBLK = E // 128
NSUB = 16
BPT = 78
STRIPE = 624


def _rb(i):
    return (i, 0)


def _const(i):
    return (0, 0)


def _embed(x_raw, emb):
    ins = [x_raw[nt] for nt in NTYPES]
    ws = [emb[nt]["W"] for nt in NTYPES]
    bs = [emb[nt]["b"].reshape(1, H) for nt in NTYPES]

    def body(*refs):
        xs = refs[0:4]
        wr = refs[4:8]
        br = refs[8:12]
        outs = refs[12:16]
        for i in range(4):
            outs[i][...] = (
                jnp.dot(xs[i][...], wr[i][...], preferred_element_type=F32)
                + br[i][...]
            )

    return pl.pallas_call(
        body,
        grid=(GRID,),
        in_specs=(
            [pl.BlockSpec((R_BLK, 768), _rb)] * 4
            + [pl.BlockSpec((768, H), _const)] * 4
            + [pl.BlockSpec((1, H), _const)] * 4
        ),
        out_specs=[pl.BlockSpec((R_BLK, H), _rb)] * 4,
        out_shape=[jax.ShapeDtypeStruct((N, H), F32)] * 4,
    )(*ins, *ws, *bs)


def _wself(e_cites, e_contains, e_refs):
    def body(ec, eo, er, out):
        m_doc = jnp.maximum(jnp.max(ec[0:1, :]), jnp.max(er[1:2, :]))
        m_sta = jnp.maximum(jnp.max(ec[1:2, :]), jnp.max(eo[0:1, :]))
        m_sec = jnp.max(eo[1:2, :])
        m_clm = jnp.max(er[0:1, :])
        for i, m in enumerate([m_doc, m_sta, m_sec, m_clm]):
            out[i] = jnp.where(m > 0, 1.0, 0.0).astype(F32)

    return pl.pallas_call(
        body,
        in_specs=[pl.BlockSpec((2, E), _const)] * 3,
        grid=(1,),
        out_specs=pl.BlockSpec(memory_space=pltpu.SMEM),
        out_shape=jax.ShapeDtypeStruct((4,), F32),
    )(e_cites, e_contains, e_refs)


def _pre(x, conv):
    ws_e, vs16, vd16 = [], [], []
    for (s, r, d, ek, sw) in ERELS:
        p = conv[f"{s}__{r}__{d}"]
        vs = jnp.einsum("khj,hj->kh", p["Ws"].reshape(H, NH, HD), p["as"])
        vd = jnp.einsum("khj,hj->kh", p["Wd"].reshape(H, NH, HD), p["ad"])
        ws_e.append(p["Ws"])
        vs16.append(jnp.pad(vs, ((0, 0), (0, 12))))
        vd16.append(jnp.pad(vd, ((0, 0), (0, 12))))
    ws_s, bs_s = [], []
    for nt in NTYPES:
        p = conv[f"{nt}__self_{nt}__{nt}"]
        ws_s.append(p["Ws"])
        bs_s.append(p["b"].reshape(1, H))

    src_of = [ERELS[i][0] for i in range(6)]
    dst_of = [ERELS[i][2] for i in range(6)]

    def body(*refs):
        xb = dict(zip(NTYPES, refs[0:4]))
        wse = refs[4:10]
        vsr = refs[10:16]
        vdr = refs[16:22]
        wss = refs[22:26]
        bss = refs[26:30]
        o_xs = refs[30:36]
        o_as = refs[36:42]
        o_ad = refs[42:48]
        o_self = refs[48:52]
        for i in range(6):
            xsrc = xb[src_of[i]][...]
            xdst = xb[dst_of[i]][...]
            o_xs[i][...] = jnp.dot(xsrc, wse[i][...], preferred_element_type=F32)
            o_as[i][...] = jnp.dot(xsrc, vsr[i][...], preferred_element_type=F32)
            o_ad[i][...] = jnp.dot(xdst, vdr[i][...], preferred_element_type=F32)
        for i, nt in enumerate(NTYPES):
            o_self[i][...] = (
                jnp.dot(xb[nt][...], wss[i][...], preferred_element_type=F32)
                + bss[i][...]
            )

    outs = pl.pallas_call(
        body,
        grid=(GRID,),
        in_specs=(
            [pl.BlockSpec((R_BLK, H), _rb)] * 4
            + [pl.BlockSpec((H, H), _const)] * 6
            + [pl.BlockSpec((H, 16), _const)] * 12
            + [pl.BlockSpec((H, H), _const)] * 4
            + [pl.BlockSpec((1, H), _const)] * 4
        ),
        out_specs=(
            [pl.BlockSpec((R_BLK, H), _rb)] * 6
            + [pl.BlockSpec((R_BLK, 16), _rb)] * 12
            + [pl.BlockSpec((R_BLK, H), _rb)] * 4
        ),
        out_shape=(
            [jax.ShapeDtypeStruct((N, H), F32)] * 6
            + [jax.ShapeDtypeStruct((N, 16), F32)] * 12
            + [jax.ShapeDtypeStruct((N, H), F32)] * 4
        ),
    )(*[x[nt] for nt in NTYPES], *ws_e, *vs16, *vd16, *ws_s, *bs_s)
    xs = outs[0:6]
    asc = outs[6:12]
    adc = outs[12:18]
    selfo = outs[18:22]
    return xs, asc, adc, selfo


def _sc_edges(xs, asc, adc, srcs, dsts):
    mesh = plsc.VectorSubcoreMesh(core_axis_name="c", subcore_axis_name="s")
    z128 = jnp.zeros((STRIPE, H), F32)
    z16 = jnp.zeros((STRIPE, 16), F32)

    def body(*refs):
        xsr = refs[0:6]
        ascr = refs[6:12]
        adcr = refs[12:18]
        srcr = refs[18:24]
        dstr = refs[24:30]
        z128r, z16r = refs[30], refs[31]
        numr = refs[32:38]
        denr = refs[38:44]
        (idx_s, idx_d, rows, sbuf, adr,
         acc, dacc, sem_s, sem_a, sem_g, sem_sc, sem_i) = refs[44:]

        core = lax.axis_index("c")
        sub = lax.axis_index("s")

        def ex_block(p):
            pb = p * 128

            def exe(i, _):
                for u in range(8):
                    e = i * 8 + u
                    v = sbuf[pb + e] + adr[e]
                    v = jnp.maximum(v, 0.2 * v)
                    sbuf[pb + e] = jnp.exp(v)
                return 0

            lax.fori_loop(0, 16, exe, 0)

        def mul_block(p):
            pb = p * 128

            def me(i, _):
                for u in range(8):
                    e = pb + i * 8 + u
                    ev = sbuf[e]
                    for h in range(NH):
                        bh = jnp.full((16,), ev[h], F32)
                        for q in range(2):
                            sl = pl.ds(h * 32 + q * 16, 16)
                            rows[e, sl] = rows[e, sl] * bh
                return 0

            lax.fori_loop(0, 16, me, 0)

        def stage_and_fire(r, b, p):
            ca = pltpu.async_copy(srcr[r].at[pl.ds(b * 128, 128)], idx_s, sem_i)
            cb = pltpu.async_copy(dstr[r].at[pl.ds(b * 128, 128)],
                                  idx_d.at[p], sem_i)
            ca.wait()
            cb.wait()
            pltpu.async_copy(ascr[r].at[idx_s],
                             sbuf.at[pl.ds(p * 128, 128)], sem_s)
            pltpu.async_copy(adcr[r].at[idx_d.at[p]], adr, sem_a)
            pltpu.async_copy(xsr[r].at[idx_s],
                             rows.at[pl.ds(p * 128, 128)], sem_g)

        def wait_g(p):
            pltpu.make_async_copy(z16r.at[pl.ds(0, 128)],
                                  sbuf.at[pl.ds(p * 128, 128)], sem_s).wait()
            pltpu.make_async_copy(z16r.at[pl.ds(0, 128)], adr, sem_a).wait()
            pltpu.make_async_copy(z128r.at[pl.ds(0, 128)],
                                  rows.at[pl.ds(p * 128, 128)], sem_g).wait()

        def fire_scatter(p):
            pltpu.async_copy(rows.at[pl.ds(p * 128, 128)],
                             acc.at[idx_d.at[p]], sem_sc, add=True)
            pltpu.async_copy(sbuf.at[pl.ds(p * 128, 128)],
                             dacc.at[idx_d.at[p]], sem_sc, add=True)

        def wait_scatter(p):
            pltpu.make_async_copy(rows.at[pl.ds(p * 128, 128)],
                                  acc.at[pl.ds(0, 128)], sem_sc).wait()
            pltpu.make_async_copy(sbuf.at[pl.ds(p * 128, 128)],
                                  dacc.at[pl.ds(0, 128)], sem_sc).wait()

        for r in range(6):
            @pl.when(core == r % 2)
            def _do_rel(r=r):
                rbase = sub * STRIPE
                pltpu.sync_copy(z128r, acc.at[pl.ds(rbase, STRIPE)])
                pltpu.sync_copy(z16r, dacc.at[pl.ds(rbase, STRIPE)])

                @pl.when(sub == NSUB - 1)
                def _ztail():
                    pltpu.sync_copy(z128r.at[pl.ds(0, 16)],
                                    acc.at[pl.ds(NSUB * STRIPE, 16)])
                    pltpu.sync_copy(z16r.at[pl.ds(0, 16)],
                                    dacc.at[pl.ds(NSUB * STRIPE, 16)])

                plsc.subcore_barrier()

                b0 = sub * BPT
                stage_and_fire(r, b0, 0)

                def pair(j, _):
                    bA = b0 + 2 * j
                    wait_g(0)
                    ex_block(0)

                    @pl.when(j > 0)
                    def _wsA():
                        wait_scatter(1)

                    stage_and_fire(r, bA + 1, 1)
                    mul_block(0)
                    fire_scatter(0)
                    wait_g(1)
                    ex_block(1)
                    wait_scatter(0)
                    stage_and_fire(r, bA + 2, 0)
                    mul_block(1)
                    fire_scatter(1)
                    return 0

                lax.fori_loop(0, BPT // 2, pair, 0)
                wait_scatter(1)
                wait_g(0)

                @pl.when(sub < 2)
                def _rem():
                    b = NSUB * BPT + sub
                    stage_and_fire(r, b, 0)
                    wait_g(0)
                    ex_block(0)
                    mul_block(0)
                    fire_scatter(0)
                    wait_scatter(0)

                plsc.subcore_barrier()

                pltpu.sync_copy(acc.at[pl.ds(rbase, STRIPE)],
                                numr[r].at[pl.ds(rbase, STRIPE)])
                pltpu.sync_copy(dacc.at[pl.ds(rbase, STRIPE)],
                                denr[r].at[pl.ds(rbase, STRIPE)])

                @pl.when(sub == NSUB - 1)
                def _ctail():
                    pltpu.sync_copy(acc.at[pl.ds(NSUB * STRIPE, 16)],
                                    numr[r].at[pl.ds(NSUB * STRIPE, 16)])
                    pltpu.sync_copy(dacc.at[pl.ds(NSUB * STRIPE, 16)],
                                    denr[r].at[pl.ds(NSUB * STRIPE, 16)])

    f = pl.kernel(
        body,
        mesh=mesh,
        compiler_params=pltpu.CompilerParams(use_tc_tiling_on_sc=False),
        out_type=(
            [jax.ShapeDtypeStruct((N, H), F32)] * 6
            + [jax.ShapeDtypeStruct((N, 16), F32)] * 6
        ),
        scratch_types=[
            pltpu.VMEM((128,), I32),
            pltpu.VMEM((2, 128), I32),
            pltpu.VMEM((2 * 128, H), F32),
            pltpu.VMEM((2 * 128, 16), F32),
            pltpu.VMEM((128, 16), F32),
            pltpu.VMEM_SHARED((N, H), F32),
            pltpu.VMEM_SHARED((N, 16), F32),
            pltpu.SemaphoreType.DMA,
            pltpu.SemaphoreType.DMA,
            pltpu.SemaphoreType.DMA,
            pltpu.SemaphoreType.DMA,
            pltpu.SemaphoreType.DMA,
        ],
    )
    outs = f(*xs, *asc, *adc, *srcs, *dsts, z128, z16)
    return outs[0:6], outs[6:12]


def _post(nums, dens, selfo, wself, conv, lnp):
    brels = []
    for (s, r, d, ek, sw) in ERELS:
        brels.append(conv[f"{s}__{r}__{d}"]["b"].reshape(1, H))
    lng = [lnp[nt]["g"].reshape(1, H) for nt in NTYPES]
    lnb = [lnp[nt]["b"].reshape(1, H) for nt in NTYPES]

    def body(*refs):
        numr = refs[0:6]
        denr = refs[6:12]
        selfr = refs[12:16]
        br = refs[16:22]
        gr = refs[22:26]
        b2r = refs[26:30]
        wr = refs[30]
        outs = refs[31:35]
        row = lax.broadcasted_iota(I32, (NH, H), 0)
        col = lax.broadcasted_iota(I32, (NH, H), 1) // HD
        expand = (row == col).astype(F32)
        for i, nt in enumerate(NTYPES):
            w = wr[i]
            tot = wr[i] * selfr[i][...]
            for rel in DST_RELS[nt]:
                dex = (
                    jnp.dot(denr[rel][:, 0:4], expand[:, :],
                            preferred_element_type=F32)
                    + 1e-16
                )
                tot = tot + numr[rel][...] / dex + br[rel][...]
            cnt = jnp.float32(len(DST_RELS[nt])) + w
            hh = jnp.maximum(tot / cnt, 0.0)
            mu = jnp.mean(hh, axis=-1, keepdims=True)
            var = jnp.mean((hh - mu) ** 2, axis=-1, keepdims=True)
            outs[i][...] = (
                (hh - mu) / jnp.sqrt(var + 1e-5) * gr[i][...] + b2r[i][...]
            )

    outs = pl.pallas_call(
        body,
        grid=(GRID,),
        in_specs=(
            [pl.BlockSpec((R_BLK, H), _rb)] * 6
            + [pl.BlockSpec((R_BLK, 16), _rb)] * 6
            + [pl.BlockSpec((R_BLK, H), _rb)] * 4
            + [pl.BlockSpec((1, H), _const)] * 14
            + [pl.BlockSpec(memory_space=pltpu.SMEM)]
        ),
        out_specs=[pl.BlockSpec((R_BLK, H), _rb)] * 4,
        out_shape=[jax.ShapeDtypeStruct((N, H), F32)] * 4,
    )(*nums, *dens, *selfo, *brels, *lng, *lnb, wself)
    return dict(zip(NTYPES, outs))


def _heads(claim, headp):
    names = ["citation_validity", "relevance_score", "coherence_score"]
    w1 = [headp[n]["W1"] for n in names]
    b1 = [headp[n]["b1"].reshape(1, H) for n in names]
    w2 = [jnp.pad(headp[n]["W2"], ((0, 0), (0, 7))) for n in names]
    b2 = [jnp.pad(headp[n]["b2"], (0, 7)).reshape(1, 8) for n in names]

    def body(c, *refs):
        w1r = refs[0:3]
        b1r = refs[3:6]
        w2r = refs[6:9]
        b2r = refs[9:12]
        outs = refs[12:15]
        cb = c[...]
        for i in range(3):
            hh = jnp.maximum(
                jnp.dot(cb, w1r[i][...], preferred_element_type=F32)
                + b1r[i][...],
                0.0,
            )
            z = jnp.dot(hh, w2r[i][...], preferred_element_type=F32) + b2r[i][...]
            outs[i][...] = 1.0 / (1.0 + jnp.exp(-z))

    outs = pl.pallas_call(
        body,
        grid=(GRID,),
        in_specs=(
            [pl.BlockSpec((R_BLK, H), _rb)]
            + [pl.BlockSpec((H, H), _const)] * 3
            + [pl.BlockSpec((1, H), _const)] * 3
            + [pl.BlockSpec((H, 8), _const)] * 3
            + [pl.BlockSpec((1, 8), _const)] * 3
        ),
        out_specs=[pl.BlockSpec((R_BLK, 8), _rb)] * 3,
        out_shape=[jax.ShapeDtypeStruct((N, 8), F32)] * 3,
    )(claim, *w1, *b1, *w2, *b2)
    return [o[:, 0:1] for o in outs]


def _sc_edges_xla(xs, asc, adc, srcs, dsts):
    nums, dens = [], []
    for r in range(6):
        al = asc[r][srcs[r]] + adc[r][dsts[r]]
        al = jnp.maximum(al, 0.2 * al)
        ex = jnp.exp(al)
        dens.append(jax.ops.segment_sum(ex, dsts[r], num_segments=N))
        msg = xs[r][srcs[r]].reshape(-1, NH, HD) * ex[:, 0:4][:, :, None]
        nums.append(jax.ops.segment_sum(msg, dsts[r],
                                        num_segments=N).reshape(N, H))
    return nums, dens


def kernel(x_document, x_statute, x_section, x_claim,
           edge_index_cites, edge_index_contains, edge_index_references,
           params):
    earr = {"cites": edge_index_cites, "contains": edge_index_contains,
            "references": edge_index_references}
    srcs, dsts = [], []
    for (s, r, d, ek, sw) in ERELS:
        ei = earr[ek]
        srcs.append(ei[1] if sw else ei[0])
        dsts.append(ei[0] if sw else ei[1])

    x_raw = {"document": x_document, "statute": x_statute,
             "section": x_section, "claim": x_claim}
    xs0 = _embed(x_raw, params["emb"])
    x = dict(zip(NTYPES, xs0))
    wself = _wself(edge_index_cites, edge_index_contains,
                   edge_index_references)

    for layer in range(2):
        conv = params["conv"][layer]
        xs, asc, adc, selfo = _pre(x, conv)
        nums, dens = _sc_edges(xs, asc, adc, srcs, dsts)
        x = _post(nums, dens, selfo, wself, conv, params["ln"][layer])

    claim = x["claim"]
    h1, h2, h3 = _heads(claim, params["heads"])
    return (claim, h1, h2, h3)
